# Initial kernel scaffold; baseline (speedup 1.0000x reference)
import jax
import jax.numpy as jnp
from jax.experimental import pallas as pl


def kernel(hidden_features, current_epoch, sW1, sb1, sW2, sb2, sW3, sb3, fW1, fb1, fW2, fb2):
    raise NotImplementedError("write your pallas kernel here")


# speedup vs baseline: 2.3916x; 2.3916x over previous
import functools

import jax
import jax.numpy as jnp
from jax import lax
from jax.experimental import pallas as pl

N = 10000
K = 16
NPAD = 10240


def _fold8(x):
    y = x
    while y.shape[1] > 1:
        w = y.shape[1] // 2
        y = y[:, :w] + y[:, w:]
    return y


def _mlp_body(h_ref, w1_ref, b1_ref, w2_ref, b2_ref, w3_ref, b3_ref, s_ref, d2_ref):
    h = h_ref[...]
    s1 = jnp.maximum(jnp.dot(h, w1_ref[...], preferred_element_type=jnp.float32) + b1_ref[...], 0.0)
    s2 = jnp.maximum(jnp.dot(s1, w2_ref[...], preferred_element_type=jnp.float32) + b2_ref[...], 0.0)
    s3 = jnp.dot(s2, w3_ref[...], preferred_element_type=jnp.float32) + b3_ref[...]
    nrm = jnp.sqrt(_fold8(s3 * s3))
    s = s3 / jnp.maximum(nrm, 1e-12)
    s_ref[...] = s
    d2_ref[...] = _fold8(s * s)


def _spatial_mlp(h, sW1, sb1, sW2, sb2, sW3, sb3):
    tile = 1000
    return pl.pallas_call(
        _mlp_body,
        grid=(N // tile,),
        in_specs=[
            pl.BlockSpec((tile, 257), lambda i: (i, 0)),
            pl.BlockSpec((257, 256), lambda i: (0, 0)),
            pl.BlockSpec((1, 256), lambda i: (0, 0)),
            pl.BlockSpec((256, 256), lambda i: (0, 0)),
            pl.BlockSpec((1, 256), lambda i: (0, 0)),
            pl.BlockSpec((256, 8), lambda i: (0, 0)),
            pl.BlockSpec((1, 8), lambda i: (0, 0)),
        ],
        out_specs=[
            pl.BlockSpec((tile, 8), lambda i: (i, 0)),
            pl.BlockSpec((tile, 1), lambda i: (i, 0)),
        ],
        out_shape=[
            jax.ShapeDtypeStruct((N, 8), jnp.float32),
            jax.ShapeDtypeStruct((N, 1), jnp.float32),
        ],
    )(h, sW1, sb1.reshape(1, -1), sW2, sb2.reshape(1, -1), sW3, sb3.reshape(1, -1))


def _knn_body(sc_ref, st_ref, d2c_ref, d2r_ref, idx_ref, w_ref):
    rt = sc_ref.shape[0]
    dotB = jnp.dot(sc_ref[...], st_ref[...], preferred_element_type=jnp.float32)
    Dm = d2c_ref[...] + d2r_ref[...] - 2.0 * dotB
    dotF = jnp.dot(sc_ref[...], st_ref[...], preferred_element_type=jnp.float32,
                   precision=lax.Precision.HIGHEST)
    DmF = d2c_ref[...] + d2r_ref[...] - 2.0 * dotF
    col = lax.broadcasted_iota(jnp.int32, (rt, N), 1)
    inf = jnp.float32(jnp.inf)
    x = Dm
    idxs, dfs = [], []
    for _ in range(K):
        m = jnp.min(x, axis=1, keepdims=True)
        cand = jnp.where(x <= m, col, jnp.int32(2 ** 30))
        j = jnp.min(cand, axis=1)
        sel = col == j[:, None]
        df = jnp.min(jnp.where(sel, DmF, inf), axis=1)
        x = jnp.where(sel, inf, x)
        idxs.append(j)
        dfs.append(df)
    idx_ref[...] = jnp.stack(idxs, axis=1)
    w_ref[...] = jnp.exp(-1.0 * jnp.stack(dfs, axis=1) / jnp.float32(0.09))


def _knn(s, d2):
    rt = 256
    s_pad = jnp.pad(s, ((0, NPAD - N), (0, 0)))
    d2_pad = jnp.pad(d2, ((0, NPAD - N), (0, 0)))
    idx, w = pl.pallas_call(
        _knn_body,
        grid=(NPAD // rt,),
        in_specs=[
            pl.BlockSpec((rt, 8), lambda i: (i, 0)),
            pl.BlockSpec((8, N), lambda i: (0, 0)),
            pl.BlockSpec((rt, 1), lambda i: (i, 0)),
            pl.BlockSpec((1, N), lambda i: (0, 0)),
        ],
        out_specs=[
            pl.BlockSpec((rt, K), lambda i: (i, 0)),
            pl.BlockSpec((rt, K), lambda i: (i, 0)),
        ],
        out_shape=[
            jax.ShapeDtypeStruct((NPAD, K), jnp.int32),
            jax.ShapeDtypeStruct((NPAD, K), jnp.float32),
        ],
    )(s_pad, s.T, d2_pad, d2.reshape(1, -1))
    return idx[:N], w[:N]


def kernel(hidden_features, current_epoch, sW1, sb1, sW2, sb2, sW3, sb3, fW1, fb1, fW2, fb2):
    h = jnp.concatenate([hidden_features, hidden_features.mean(axis=1, keepdims=True)], axis=-1)
    s, d2 = _spatial_mlp(h, sW1, sb1, sW2, sb2, sW3, sb3)
    idx, _w = _knn(s, d2)

    start = idx.reshape(-1).astype(jnp.int32)
    end = jnp.repeat(jnp.arange(N, dtype=jnp.int32), K)
    order = jnp.argsort(start * N + end)
    start = start[order]
    end = end[order]
    edge_index = jnp.stack([start, end], axis=0)

    d = jnp.sum((s[start] - s[end]) ** 2, axis=-1)
    w = jnp.exp(-1.0 * d / (0.3 * 0.3))
    agg = jax.ops.segment_sum(h[start] * w[:, None], end, num_segments=N)
    cat = jnp.concatenate([agg, h], axis=-1)
    out = jax.nn.relu(cat @ fW1 + fb1)
    out = out @ fW2 + fb2
    return (out, edge_index, s, 1.0)


# speedup vs baseline: 3.4659x; 1.4492x over previous
import functools

import jax
import jax.numpy as jnp
from jax import lax
from jax.experimental import pallas as pl
from jax.experimental.pallas import tpu as pltpu
from jax.experimental.pallas import tpu_sc as plsc

N = 10000
K = 16
NPAD = 10240
HW = 272
CPT = NPAD // 32


def _fold(x):
    y = x
    while y.shape[1] > 1:
        w = y.shape[1] // 2
        y = y[:, :w] + y[:, w:]
    return y


def _mlp_body(h_ref, w1_ref, b1_ref, w2_ref, b2_ref, w3_ref, b3_ref, s_ref, d2_ref):
    h = h_ref[...]
    s1 = jnp.maximum(jnp.dot(h, w1_ref[...], preferred_element_type=jnp.float32) + b1_ref[...], 0.0)
    s2 = jnp.maximum(jnp.dot(s1, w2_ref[...], preferred_element_type=jnp.float32) + b2_ref[...], 0.0)
    s3 = jnp.dot(s2, w3_ref[...], preferred_element_type=jnp.float32) + b3_ref[...]
    nrm = jnp.sqrt(_fold(s3 * s3))
    s = s3 / jnp.maximum(nrm, 1e-12)
    s_ref[...] = s
    d2_ref[...] = _fold(s * s)


def _spatial_mlp(h, sW1, sb1, sW2, sb2, sW3, sb3):
    tile = 1000
    return pl.pallas_call(
        _mlp_body,
        grid=(N // tile,),
        in_specs=[
            pl.BlockSpec((tile, 257), lambda i: (i, 0)),
            pl.BlockSpec((257, 256), lambda i: (0, 0)),
            pl.BlockSpec((1, 256), lambda i: (0, 0)),
            pl.BlockSpec((256, 256), lambda i: (0, 0)),
            pl.BlockSpec((1, 256), lambda i: (0, 0)),
            pl.BlockSpec((256, 8), lambda i: (0, 0)),
            pl.BlockSpec((1, 8), lambda i: (0, 0)),
        ],
        out_specs=[
            pl.BlockSpec((tile, 8), lambda i: (i, 0)),
            pl.BlockSpec((tile, 1), lambda i: (i, 0)),
        ],
        out_shape=[
            jax.ShapeDtypeStruct((N, 8), jnp.float32),
            jax.ShapeDtypeStruct((N, 1), jnp.float32),
        ],
    )(h, sW1, sb1.reshape(1, -1), sW2, sb2.reshape(1, -1), sW3, sb3.reshape(1, -1))


def _knn_body(sc_ref, st_ref, d2c_ref, d2r_ref, idx_ref, w_ref):
    rt = sc_ref.shape[0]
    dotB = jnp.dot(sc_ref[...], st_ref[...], preferred_element_type=jnp.float32)
    Dm = d2c_ref[...] + d2r_ref[...] - 2.0 * dotB
    dotF = jnp.dot(sc_ref[...], st_ref[...], preferred_element_type=jnp.float32,
                   precision=lax.Precision.HIGHEST)
    DmF = d2c_ref[...] + d2r_ref[...] - 2.0 * dotF
    col = lax.broadcasted_iota(jnp.int32, (rt, N), 1)
    inf = jnp.float32(jnp.inf)
    x = Dm
    idxs, dfs = [], []
    for _ in range(K):
        m = jnp.min(x, axis=1, keepdims=True)
        cand = jnp.where(x <= m, col, jnp.int32(2 ** 30))
        j = jnp.min(cand, axis=1)
        sel = col == j[:, None]
        df = jnp.min(jnp.where(sel, DmF, inf), axis=1)
        x = jnp.where(sel, inf, x)
        idxs.append(j)
        dfs.append(df)
    idx_ref[...] = jnp.stack(idxs, axis=1)
    w_ref[...] = jnp.exp(jnp.stack(dfs, axis=1) * jnp.float32(-1.0 / 0.09))


def _knn(s, d2):
    rt = 256
    s_pad = jnp.pad(s, ((0, NPAD - N), (0, 0)))
    d2_pad = jnp.pad(d2, ((0, NPAD - N), (0, 0)))
    idx, w = pl.pallas_call(
        _knn_body,
        grid=(NPAD // rt,),
        in_specs=[
            pl.BlockSpec((rt, 8), lambda i: (i, 0)),
            pl.BlockSpec((8, N), lambda i: (0, 0)),
            pl.BlockSpec((rt, 1), lambda i: (i, 0)),
            pl.BlockSpec((1, N), lambda i: (0, 0)),
        ],
        out_specs=[pl.BlockSpec((rt, K), lambda i: (i, 0)),
                   pl.BlockSpec((rt, K), lambda i: (i, 0))],
        out_shape=[jax.ShapeDtypeStruct((NPAD, K), jnp.int32),
                   jax.ShapeDtypeStruct((NPAD, K), jnp.float32)],
    )(s_pad, s.T, d2_pad, d2.reshape(1, -1))
    return idx, w


def _rotsum16(v):
    io = lax.broadcasted_iota(jnp.int32, (16,), 0)
    for sh in (8, 4, 2, 1):
        v = v + jnp.take(v, (io + sh) % 16)
    return v


CHUNK = 64


def _make_sc_agg():
    mesh = plsc.VectorSubcoreMesh(core_axis_name="c", subcore_axis_name="s")

    @functools.partial(
        pl.kernel,
        mesh=mesh,
        out_type=jax.ShapeDtypeStruct((NPAD, HW), jnp.float32),
        compiler_params=pltpu.CompilerParams(
            needs_layout_passes=False, use_tc_tiling_on_sc=False),
        scratch_types=[
            pltpu.VMEM((CPT, K), jnp.int32),
            pltpu.VMEM((CPT, K), jnp.float32),
            pltpu.VMEM((N, 1), jnp.float32),
            pltpu.VMEM((K, 256), jnp.float32),
            pltpu.VMEM((CHUNK, HW), jnp.float32),
            pltpu.SemaphoreType.DMA,
        ],
    )
    def agg_kernel(hid_hbm, idx_hbm, w_hbm, hm_hbm, out_hbm,
                   idxv, wvb, hmv, rowsv, outv, sem):
        nc = 2
        wid = lax.axis_index("s") * nc + lax.axis_index("c")
        base = wid * CPT
        pltpu.sync_copy(idx_hbm.at[pl.ds(base, CPT)], idxv)
        pltpu.sync_copy(w_hbm.at[pl.ds(base, CPT)], wvb)
        pltpu.sync_copy(hm_hbm, hmv)

        for ch in range(CPT // CHUNK):
            def body(t, _, ch=ch):
                tt = ch * CHUNK + t
                iv = idxv[tt]
                pltpu.async_copy(hid_hbm.at[iv], rowsv, sem).wait()
                wv = wvb[tt]
                hmg = plsc.load_gather(hmv, [iv, jnp.zeros((16,), jnp.int32)])
                aggm = _rotsum16(wv * hmg)
                accs = [jnp.zeros((16,), jnp.float32) for _ in range(16)]
                for k in range(K):
                    wk = jnp.take(wv, jnp.full((16,), k, jnp.int32))
                    for c in range(16):
                        accs[c] = accs[c] + wk * rowsv[k, pl.ds(c * 16, 16)]
                for c in range(16):
                    outv[t, pl.ds(c * 16, 16)] = accs[c]
                outv[t, pl.ds(256, 16)] = aggm
                return _

            lax.fori_loop(0, CHUNK, body, 0)
            pltpu.sync_copy(outv, out_hbm.at[pl.ds(base + ch * CHUNK, CHUNK)])

    return agg_kernel


def _sc_aggregate(hidden, idx, w, hm):
    return _make_sc_agg()(hidden, idx, w, hm)


def _final_body(agg_ref, hid_ref, hm_ref, a_ref, am_ref, b_ref, bm_ref,
                fb1_ref, w2_ref, fb2_ref, o_ref):
    agg = agg_ref[...]
    pre = jnp.dot(agg[:, 0:256], a_ref[...], preferred_element_type=jnp.float32)
    pre = pre + agg[:, 256:257] * am_ref[...]
    pre = pre + jnp.dot(hid_ref[...], b_ref[...], preferred_element_type=jnp.float32)
    pre = pre + hm_ref[...] * bm_ref[...]
    act = jnp.maximum(pre + fb1_ref[...], 0.0)
    o_ref[...] = jnp.dot(act, w2_ref[...], preferred_element_type=jnp.float32) + fb2_ref[...]


def _final_mlp(agg, hidden, hm, fW1, fb1, fW2, fb2):
    tile = 1000
    return pl.pallas_call(
        _final_body,
        grid=(N // tile,),
        in_specs=[
            pl.BlockSpec((tile, HW), lambda i: (i, 0)),
            pl.BlockSpec((tile, 256), lambda i: (i, 0)),
            pl.BlockSpec((tile, 1), lambda i: (i, 0)),
            pl.BlockSpec((256, 256), lambda i: (0, 0)),
            pl.BlockSpec((1, 256), lambda i: (0, 0)),
            pl.BlockSpec((256, 256), lambda i: (0, 0)),
            pl.BlockSpec((1, 256), lambda i: (0, 0)),
            pl.BlockSpec((1, 256), lambda i: (0, 0)),
            pl.BlockSpec((256, 256), lambda i: (0, 0)),
            pl.BlockSpec((1, 256), lambda i: (0, 0)),
        ],
        out_specs=pl.BlockSpec((tile, 256), lambda i: (i, 0)),
        out_shape=jax.ShapeDtypeStruct((N, 256), jnp.float32),
    )(agg[:N], hidden, hm,
      fW1[0:256], fW1[256:257], fW1[257:513], fW1[513:514],
      fb1.reshape(1, -1), fW2, fb2.reshape(1, -1))


def kernel(hidden_features, current_epoch, sW1, sb1, sW2, sb2, sW3, sb3, fW1, fb1, fW2, fb2):
    hm = hidden_features.mean(axis=1, keepdims=True)
    h = jnp.concatenate([hidden_features, hm], axis=-1)
    s, d2 = _spatial_mlp(h, sW1, sb1, sW2, sb2, sW3, sb3)
    idx, w = _knn(s, d2)

    agg = _sc_aggregate(hidden_features, idx, w, hm)

    out = _final_mlp(agg, hidden_features, hm, fW1, fb1, fW2, fb2)

    idx10 = idx[:N]
    start = idx10.reshape(-1).astype(jnp.int32)
    end = jnp.repeat(jnp.arange(N, dtype=jnp.int32), K)
    order = jnp.argsort(start * N + end)
    edge_index = jnp.stack([start[order], end[order]], axis=0)
    return (out, edge_index, s, 1.0)


# speedup vs baseline: 4.4816x; 1.2930x over previous
import functools

import jax
import jax.numpy as jnp
from jax import lax
from jax.experimental import pallas as pl
from jax.experimental.pallas import tpu as pltpu
from jax.experimental.pallas import tpu_sc as plsc

N = 10000
K = 16
NPAD = 10240
HW = 272
CPT = NPAD // 32


def _fold(x):
    y = x
    while y.shape[1] > 1:
        w = y.shape[1] // 2
        y = y[:, :w] + y[:, w:]
    return y


def _mlp_body(h_ref, w1_ref, b1_ref, w2_ref, b2_ref, w3_ref, b3_ref, s_ref, d2_ref):
    h = h_ref[...]
    s1 = jnp.maximum(jnp.dot(h, w1_ref[...], preferred_element_type=jnp.float32) + b1_ref[...], 0.0)
    s2 = jnp.maximum(jnp.dot(s1, w2_ref[...], preferred_element_type=jnp.float32) + b2_ref[...], 0.0)
    s3 = jnp.dot(s2, w3_ref[...], preferred_element_type=jnp.float32) + b3_ref[...]
    nrm = jnp.sqrt(_fold(s3 * s3))
    s = s3 / jnp.maximum(nrm, 1e-12)
    s_ref[...] = s
    d2_ref[...] = _fold(s * s)


def _spatial_mlp(h, sW1, sb1, sW2, sb2, sW3, sb3):
    tile = 1000
    return pl.pallas_call(
        _mlp_body,
        grid=(N // tile,),
        in_specs=[
            pl.BlockSpec((tile, 257), lambda i: (i, 0)),
            pl.BlockSpec((257, 256), lambda i: (0, 0)),
            pl.BlockSpec((1, 256), lambda i: (0, 0)),
            pl.BlockSpec((256, 256), lambda i: (0, 0)),
            pl.BlockSpec((1, 256), lambda i: (0, 0)),
            pl.BlockSpec((256, 8), lambda i: (0, 0)),
            pl.BlockSpec((1, 8), lambda i: (0, 0)),
        ],
        out_specs=[
            pl.BlockSpec((tile, 8), lambda i: (i, 0)),
            pl.BlockSpec((tile, 1), lambda i: (i, 0)),
        ],
        out_shape=[
            jax.ShapeDtypeStruct((N, 8), jnp.float32),
            jax.ShapeDtypeStruct((N, 1), jnp.float32),
        ],
    )(h, sW1, sb1.reshape(1, -1), sW2, sb2.reshape(1, -1), sW3, sb3.reshape(1, -1))


def _knn_body(sc_ref, st_ref, d2c_ref, d2r_ref, idx_ref):
    rt = sc_ref.shape[0]
    dotB = jnp.dot(sc_ref[...], st_ref[...], preferred_element_type=jnp.float32)
    Dm = d2c_ref[...] + d2r_ref[...] - 2.0 * dotB
    col = lax.broadcasted_iota(jnp.int32, (rt, N), 1)
    inf = jnp.float32(jnp.inf)
    x = Dm
    idxs = []
    for _ in range(K):
        m = jnp.min(x, axis=1, keepdims=True)
        cand = jnp.where(x <= m, col, jnp.int32(2 ** 30))
        j = jnp.min(cand, axis=1)
        x = jnp.where(col == j[:, None], inf, x)
        idxs.append(j)
    idx_ref[...] = jnp.stack(idxs, axis=1)


def _knn(s, d2):
    rt = 256
    s_pad = jnp.pad(s, ((0, NPAD - N), (0, 0)))
    d2_pad = jnp.pad(d2, ((0, NPAD - N), (0, 0)))
    idx = pl.pallas_call(
        _knn_body,
        grid=(NPAD // rt,),
        in_specs=[
            pl.BlockSpec((rt, 8), lambda i: (i, 0)),
            pl.BlockSpec((8, N), lambda i: (0, 0)),
            pl.BlockSpec((rt, 1), lambda i: (i, 0)),
            pl.BlockSpec((1, N), lambda i: (0, 0)),
        ],
        out_specs=pl.BlockSpec((rt, K), lambda i: (i, 0)),
        out_shape=jax.ShapeDtypeStruct((NPAD, K), jnp.int32),
    )(s_pad, s.T, d2_pad, d2.reshape(1, -1))
    return idx


def _rotsum16(v):
    io = lax.broadcasted_iota(jnp.int32, (16,), 0)
    for sh in (8, 4, 2, 1):
        v = v + jnp.take(v, (io + sh) % 16)
    return v


CHUNK = 64


def _make_sc_agg():
    mesh = plsc.VectorSubcoreMesh(core_axis_name="c", subcore_axis_name="s")

    @functools.partial(
        pl.kernel,
        mesh=mesh,
        out_type=jax.ShapeDtypeStruct((NPAD, HW), jnp.float32),
        compiler_params=pltpu.CompilerParams(
            needs_layout_passes=False, use_tc_tiling_on_sc=False),
        scratch_types=[
            pltpu.VMEM((CPT, K), jnp.int32),
            pltpu.VMEM((CPT, 16), jnp.float32),
            pltpu.VMEM((N, 1), jnp.float32),
            pltpu.VMEM((K, 256), jnp.float32),
            pltpu.VMEM((K, 128), jnp.float32),
            pltpu.VMEM((CHUNK, HW), jnp.float32),
            pltpu.SemaphoreType.DMA,
            pltpu.SemaphoreType.DMA,
        ],
    )
    def agg_kernel(hid_hbm, idx_hbm, s128_hbm, spad_hbm, hm_hbm, out_hbm,
                   idxv, sv, hmv, rowsv, rows2v, outv, sem, sem2):
        nc = 2
        wid = lax.axis_index("s") * nc + lax.axis_index("c")
        base = wid * CPT
        pltpu.sync_copy(idx_hbm.at[pl.ds(base, CPT)], idxv)
        pltpu.sync_copy(spad_hbm.at[pl.ds(base, CPT)], sv)
        pltpu.sync_copy(hm_hbm, hmv)
        io16 = lax.broadcasted_iota(jnp.int32, (16,), 0)

        for ch in range(CPT // CHUNK):
            def body(t, _, ch=ch):
                tt = ch * CHUNK + t
                iv = idxv[tt]
                cp1 = pltpu.async_copy(hid_hbm.at[iv], rowsv, sem)
                cp2 = pltpu.async_copy(s128_hbm.at[iv], rows2v, sem2)
                cp1.wait()
                cp2.wait()
                s_i = sv[tt]
                hmg = plsc.load_gather(hmv, [iv, jnp.zeros((16,), jnp.int32)])
                accs = [jnp.zeros((16,), jnp.float32) for _ in range(16)]
                wvec = jnp.zeros((16,), jnp.float32)
                for k in range(K):
                    dv = rows2v[k, pl.ds(0, 16)] - s_i
                    dtot = _rotsum16(dv * dv)
                    wk = jnp.exp(dtot * jnp.float32(-1.0 / 0.09))
                    wvec = jnp.where(io16 == k, wk, wvec)
                    for c in range(16):
                        accs[c] = accs[c] + wk * rowsv[k, pl.ds(c * 16, 16)]
                aggm = _rotsum16(wvec * hmg)
                for c in range(16):
                    outv[t, pl.ds(c * 16, 16)] = accs[c]
                outv[t, pl.ds(256, 16)] = aggm
                return _

            lax.fori_loop(0, CHUNK, body, 0)
            pltpu.sync_copy(outv, out_hbm.at[pl.ds(base + ch * CHUNK, CHUNK)])

    return agg_kernel


def _sc_aggregate(hidden, idx, s, s_pad16, hm):
    s128 = jnp.pad(s, ((0, 0), (0, 120)))
    return _make_sc_agg()(hidden, idx, s128, s_pad16, hm)


def _final_body(agg_ref, hid_ref, hm_ref, a_ref, am_ref, b_ref, bm_ref,
                fb1_ref, w2_ref, fb2_ref, o_ref):
    agg = agg_ref[...]
    pre = jnp.dot(agg[:, 0:256], a_ref[...], preferred_element_type=jnp.float32)
    pre = pre + agg[:, 256:257] * am_ref[...]
    pre = pre + jnp.dot(hid_ref[...], b_ref[...], preferred_element_type=jnp.float32)
    pre = pre + hm_ref[...] * bm_ref[...]
    act = jnp.maximum(pre + fb1_ref[...], 0.0)
    o_ref[...] = jnp.dot(act, w2_ref[...], preferred_element_type=jnp.float32) + fb2_ref[...]


def _final_mlp(agg, hidden, hm, fW1, fb1, fW2, fb2):
    tile = 1000
    return pl.pallas_call(
        _final_body,
        grid=(N // tile,),
        in_specs=[
            pl.BlockSpec((tile, HW), lambda i: (i, 0)),
            pl.BlockSpec((tile, 256), lambda i: (i, 0)),
            pl.BlockSpec((tile, 1), lambda i: (i, 0)),
            pl.BlockSpec((256, 256), lambda i: (0, 0)),
            pl.BlockSpec((1, 256), lambda i: (0, 0)),
            pl.BlockSpec((256, 256), lambda i: (0, 0)),
            pl.BlockSpec((1, 256), lambda i: (0, 0)),
            pl.BlockSpec((1, 256), lambda i: (0, 0)),
            pl.BlockSpec((256, 256), lambda i: (0, 0)),
            pl.BlockSpec((1, 256), lambda i: (0, 0)),
        ],
        out_specs=pl.BlockSpec((tile, 256), lambda i: (i, 0)),
        out_shape=jax.ShapeDtypeStruct((N, 256), jnp.float32),
    )(agg[:N], hidden, hm,
      fW1[0:256], fW1[256:257], fW1[257:513], fW1[513:514],
      fb1.reshape(1, -1), fW2, fb2.reshape(1, -1))


def kernel(hidden_features, current_epoch, sW1, sb1, sW2, sb2, sW3, sb3, fW1, fb1, fW2, fb2):
    hm = hidden_features.mean(axis=1, keepdims=True)
    h = jnp.concatenate([hidden_features, hm], axis=-1)
    s, d2 = _spatial_mlp(h, sW1, sb1, sW2, sb2, sW3, sb3)
    idx = _knn(s, d2)

    s_pad16 = jnp.pad(s, ((0, NPAD - N), (0, 8)))
    agg = _sc_aggregate(hidden_features, idx, s, s_pad16, hm)

    out = _final_mlp(agg, hidden_features, hm, fW1, fb1, fW2, fb2)

    idx10 = idx[:N]
    start = idx10.reshape(-1).astype(jnp.int32)
    end = jnp.repeat(jnp.arange(N, dtype=jnp.int32), K)
    order = jnp.argsort(start * N + end)
    edge_index = jnp.stack([start[order], end[order]], axis=0)
    return (out, edge_index, s, 1.0)


# speedup vs baseline: 4.4824x; 1.0002x over previous
import functools

import jax
import jax.numpy as jnp
from jax import lax
from jax.experimental import pallas as pl
from jax.experimental.pallas import tpu as pltpu
from jax.experimental.pallas import tpu_sc as plsc

N = 10000
K = 16
NPAD = 10240
HW = 272
CPT = NPAD // 32


def _fold(x):
    y = x
    while y.shape[1] > 1:
        w = y.shape[1] // 2
        y = y[:, :w] + y[:, w:]
    return y


def _mlp_body(h_ref, w1_ref, b1_ref, w2_ref, b2_ref, w3_ref, b3_ref, s_ref, d2_ref):
    h = h_ref[...]
    s1 = jnp.maximum(jnp.dot(h, w1_ref[...], preferred_element_type=jnp.float32) + b1_ref[...], 0.0)
    s2 = jnp.maximum(jnp.dot(s1, w2_ref[...], preferred_element_type=jnp.float32) + b2_ref[...], 0.0)
    s3 = jnp.dot(s2, w3_ref[...], preferred_element_type=jnp.float32) + b3_ref[...]
    nrm = jnp.sqrt(_fold(s3 * s3))
    s = s3 / jnp.maximum(nrm, 1e-12)
    s_ref[...] = s
    d2_ref[...] = _fold(s * s)


def _spatial_mlp(h, sW1, sb1, sW2, sb2, sW3, sb3):
    tile = 1000
    return pl.pallas_call(
        _mlp_body,
        grid=(N // tile,),
        in_specs=[
            pl.BlockSpec((tile, 257), lambda i: (i, 0)),
            pl.BlockSpec((257, 256), lambda i: (0, 0)),
            pl.BlockSpec((1, 256), lambda i: (0, 0)),
            pl.BlockSpec((256, 256), lambda i: (0, 0)),
            pl.BlockSpec((1, 256), lambda i: (0, 0)),
            pl.BlockSpec((256, 8), lambda i: (0, 0)),
            pl.BlockSpec((1, 8), lambda i: (0, 0)),
        ],
        out_specs=[
            pl.BlockSpec((tile, 8), lambda i: (i, 0)),
            pl.BlockSpec((tile, 1), lambda i: (i, 0)),
        ],
        out_shape=[
            jax.ShapeDtypeStruct((N, 8), jnp.float32),
            jax.ShapeDtypeStruct((N, 1), jnp.float32),
        ],
    )(h, sW1, sb1.reshape(1, -1), sW2, sb2.reshape(1, -1), sW3, sb3.reshape(1, -1))


def _knn_body(sc_ref, st_ref, d2c_ref, d2r_ref, idx_ref):
    rt = sc_ref.shape[0]
    dotB = jnp.dot(sc_ref[...], st_ref[...], preferred_element_type=jnp.float32)
    Dm = d2c_ref[...] + d2r_ref[...] - 2.0 * dotB
    col = lax.broadcasted_iota(jnp.int32, (rt, N), 1)
    inf = jnp.float32(jnp.inf)
    x = Dm
    m = jnp.min(x, axis=1, keepdims=True)
    idxs = []
    for k in range(K):
        cand = jnp.where(x <= m, col, jnp.int32(2 ** 30))
        j = jnp.min(cand, axis=1)
        if k < K - 1:
            x = jnp.where(col == j[:, None], inf, x)
            m = jnp.min(x, axis=1, keepdims=True)
        idxs.append(j)
    idx_ref[...] = jnp.stack(idxs, axis=1)


def _knn(s, d2):
    rt = 256
    s_pad = jnp.pad(s, ((0, NPAD - N), (0, 0)))
    d2_pad = jnp.pad(d2, ((0, NPAD - N), (0, 0)))
    idx = pl.pallas_call(
        _knn_body,
        grid=(NPAD // rt,),
        in_specs=[
            pl.BlockSpec((rt, 8), lambda i: (i, 0)),
            pl.BlockSpec((8, N), lambda i: (0, 0)),
            pl.BlockSpec((rt, 1), lambda i: (i, 0)),
            pl.BlockSpec((1, N), lambda i: (0, 0)),
        ],
        out_specs=pl.BlockSpec((rt, K), lambda i: (i, 0)),
        out_shape=jax.ShapeDtypeStruct((NPAD, K), jnp.int32),
    )(s_pad, s.T, d2_pad, d2.reshape(1, -1))
    return idx


def _rotsum16(v):
    io = lax.broadcasted_iota(jnp.int32, (16,), 0)
    for sh in (8, 4, 2, 1):
        v = v + jnp.take(v, (io + sh) % 16)
    return v


CHUNK = 64


def _make_sc_agg():
    mesh = plsc.VectorSubcoreMesh(core_axis_name="c", subcore_axis_name="s")

    @functools.partial(
        pl.kernel,
        mesh=mesh,
        out_type=jax.ShapeDtypeStruct((NPAD, HW), jnp.float32),
        compiler_params=pltpu.CompilerParams(
            needs_layout_passes=False, use_tc_tiling_on_sc=False),
        scratch_types=[
            pltpu.VMEM((CPT, K), jnp.int32),
            pltpu.VMEM((CPT, 16), jnp.float32),
            pltpu.VMEM((N, 1), jnp.float32),
            pltpu.VMEM((K, 256), jnp.float32),
            pltpu.VMEM((K, 128), jnp.float32),
            pltpu.VMEM((CHUNK, HW), jnp.float32),
            pltpu.SemaphoreType.DMA,
            pltpu.SemaphoreType.DMA,
        ],
    )
    def agg_kernel(hid_hbm, idx_hbm, s128_hbm, spad_hbm, hm_hbm, out_hbm,
                   idxv, sv, hmv, rowsv, rows2v, outv, sem, sem2):
        nc = 2
        wid = lax.axis_index("s") * nc + lax.axis_index("c")
        base = wid * CPT
        pltpu.sync_copy(idx_hbm.at[pl.ds(base, CPT)], idxv)
        pltpu.sync_copy(spad_hbm.at[pl.ds(base, CPT)], sv)
        pltpu.sync_copy(hm_hbm, hmv)
        io16 = lax.broadcasted_iota(jnp.int32, (16,), 0)

        for ch in range(CPT // CHUNK):
            def body(t, _, ch=ch):
                tt = ch * CHUNK + t
                iv = idxv[tt]
                cp1 = pltpu.async_copy(hid_hbm.at[iv], rowsv, sem)
                cp2 = pltpu.async_copy(s128_hbm.at[iv], rows2v, sem2)
                cp1.wait()
                cp2.wait()
                s_i = sv[tt]
                hmg = plsc.load_gather(hmv, [iv, jnp.zeros((16,), jnp.int32)])
                accs = [jnp.zeros((16,), jnp.float32) for _ in range(16)]
                wvec = jnp.zeros((16,), jnp.float32)
                for k in range(K):
                    dv = rows2v[k, pl.ds(0, 16)] - s_i
                    dtot = _rotsum16(dv * dv)
                    wk = jnp.exp(dtot * jnp.float32(-1.0 / 0.09))
                    wvec = jnp.where(io16 == k, wk, wvec)
                    for c in range(16):
                        accs[c] = accs[c] + wk * rowsv[k, pl.ds(c * 16, 16)]
                aggm = _rotsum16(wvec * hmg)
                for c in range(16):
                    outv[t, pl.ds(c * 16, 16)] = accs[c]
                outv[t, pl.ds(256, 16)] = aggm
                return _

            lax.fori_loop(0, CHUNK, body, 0)
            pltpu.sync_copy(outv, out_hbm.at[pl.ds(base + ch * CHUNK, CHUNK)])

    return agg_kernel


def _sc_aggregate(hidden, idx, s, s_pad16, hm):
    s128 = jnp.pad(s, ((0, 0), (0, 120)))
    return _make_sc_agg()(hidden, idx, s128, s_pad16, hm)


def _final_body(agg_ref, hid_ref, hm_ref, a_ref, am_ref, b_ref, bm_ref,
                fb1_ref, w2_ref, fb2_ref, o_ref):
    agg = agg_ref[...]
    pre = jnp.dot(agg[:, 0:256], a_ref[...], preferred_element_type=jnp.float32)
    pre = pre + agg[:, 256:257] * am_ref[...]
    pre = pre + jnp.dot(hid_ref[...], b_ref[...], preferred_element_type=jnp.float32)
    pre = pre + hm_ref[...] * bm_ref[...]
    act = jnp.maximum(pre + fb1_ref[...], 0.0)
    o_ref[...] = jnp.dot(act, w2_ref[...], preferred_element_type=jnp.float32) + fb2_ref[...]


def _final_mlp(agg, hidden, hm, fW1, fb1, fW2, fb2):
    tile = 1000
    return pl.pallas_call(
        _final_body,
        grid=(N // tile,),
        in_specs=[
            pl.BlockSpec((tile, HW), lambda i: (i, 0)),
            pl.BlockSpec((tile, 256), lambda i: (i, 0)),
            pl.BlockSpec((tile, 1), lambda i: (i, 0)),
            pl.BlockSpec((256, 256), lambda i: (0, 0)),
            pl.BlockSpec((1, 256), lambda i: (0, 0)),
            pl.BlockSpec((256, 256), lambda i: (0, 0)),
            pl.BlockSpec((1, 256), lambda i: (0, 0)),
            pl.BlockSpec((1, 256), lambda i: (0, 0)),
            pl.BlockSpec((256, 256), lambda i: (0, 0)),
            pl.BlockSpec((1, 256), lambda i: (0, 0)),
        ],
        out_specs=pl.BlockSpec((tile, 256), lambda i: (i, 0)),
        out_shape=jax.ShapeDtypeStruct((N, 256), jnp.float32),
    )(agg[:N], hidden, hm,
      fW1[0:256], fW1[256:257], fW1[257:513], fW1[513:514],
      fb1.reshape(1, -1), fW2, fb2.reshape(1, -1))


def kernel(hidden_features, current_epoch, sW1, sb1, sW2, sb2, sW3, sb3, fW1, fb1, fW2, fb2):
    hm = hidden_features.mean(axis=1, keepdims=True)
    h = jnp.concatenate([hidden_features, hm], axis=-1)
    s, d2 = _spatial_mlp(h, sW1, sb1, sW2, sb2, sW3, sb3)
    idx = _knn(s, d2)

    s_pad16 = jnp.pad(s, ((0, NPAD - N), (0, 8)))
    agg = _sc_aggregate(hidden_features, idx, s, s_pad16, hm)

    out = _final_mlp(agg, hidden_features, hm, fW1, fb1, fW2, fb2)

    idx10 = idx[:N]
    start = idx10.reshape(-1).astype(jnp.int32)
    end = jnp.repeat(jnp.arange(N, dtype=jnp.int32), K)
    order = jnp.argsort(start * N + end)
    edge_index = jnp.stack([start[order], end[order]], axis=0)
    return (out, edge_index, s, 1.0)


# speedup vs baseline: 4.7102x; 1.0508x over previous
import functools

import jax
import jax.numpy as jnp
from jax import lax
from jax.experimental import pallas as pl
from jax.experimental.pallas import tpu as pltpu
from jax.experimental.pallas import tpu_sc as plsc

N = 10000
K = 16
NPAD = 10240
HW = 272
CPT = NPAD // 32


def _fold(x):
    y = x
    while y.shape[1] > 1:
        w = y.shape[1] // 2
        y = y[:, :w] + y[:, w:]
    return y


def _mlp_body(h_ref, w1_ref, b1_ref, w2_ref, b2_ref, w3_ref, b3_ref, s_ref, d2_ref):
    h = h_ref[...]
    s1 = jnp.maximum(jnp.dot(h, w1_ref[...], preferred_element_type=jnp.float32) + b1_ref[...], 0.0)
    s2 = jnp.maximum(jnp.dot(s1, w2_ref[...], preferred_element_type=jnp.float32) + b2_ref[...], 0.0)
    s3 = jnp.dot(s2, w3_ref[...], preferred_element_type=jnp.float32) + b3_ref[...]
    nrm = jnp.sqrt(_fold(s3 * s3))
    s = s3 / jnp.maximum(nrm, 1e-12)
    s_ref[...] = s
    d2_ref[...] = _fold(s * s)


def _spatial_mlp(h, sW1, sb1, sW2, sb2, sW3, sb3):
    tile = 1000
    return pl.pallas_call(
        _mlp_body,
        grid=(N // tile,),
        in_specs=[
            pl.BlockSpec((tile, 257), lambda i: (i, 0)),
            pl.BlockSpec((257, 256), lambda i: (0, 0)),
            pl.BlockSpec((1, 256), lambda i: (0, 0)),
            pl.BlockSpec((256, 256), lambda i: (0, 0)),
            pl.BlockSpec((1, 256), lambda i: (0, 0)),
            pl.BlockSpec((256, 8), lambda i: (0, 0)),
            pl.BlockSpec((1, 8), lambda i: (0, 0)),
        ],
        out_specs=[
            pl.BlockSpec((tile, 8), lambda i: (i, 0)),
            pl.BlockSpec((tile, 1), lambda i: (i, 0)),
        ],
        out_shape=[
            jax.ShapeDtypeStruct((N, 8), jnp.float32),
            jax.ShapeDtypeStruct((N, 1), jnp.float32),
        ],
    )(h, sW1, sb1.reshape(1, -1), sW2, sb2.reshape(1, -1), sW3, sb3.reshape(1, -1))


def _knn_body(sc_ref, st_ref, d2c_ref, d2r_ref, idx_ref):
    rt = sc_ref.shape[0]
    dotB = jnp.dot(sc_ref[...], st_ref[...], preferred_element_type=jnp.float32)
    Dm = d2c_ref[...] + d2r_ref[...] - 2.0 * dotB
    col = lax.broadcasted_iota(jnp.int32, (rt, N), 1)
    inf = jnp.float32(jnp.inf)
    x = Dm
    m = jnp.min(x, axis=1, keepdims=True)
    idxs = []
    for k in range(K):
        cand = jnp.where(x <= m, col, jnp.int32(2 ** 30))
        j = jnp.min(cand, axis=1)
        if k < K - 1:
            x = jnp.where(col == j[:, None], inf, x)
            m = jnp.min(x, axis=1, keepdims=True)
        idxs.append(j)
    idx_ref[...] = jnp.stack(idxs, axis=1)


def _knn(s, d2):
    rt = 256
    s_pad = jnp.pad(s, ((0, NPAD - N), (0, 0)))
    d2_pad = jnp.pad(d2, ((0, NPAD - N), (0, 0)))
    idx = pl.pallas_call(
        _knn_body,
        grid=(NPAD // rt,),
        in_specs=[
            pl.BlockSpec((rt, 8), lambda i: (i, 0)),
            pl.BlockSpec((8, N), lambda i: (0, 0)),
            pl.BlockSpec((rt, 1), lambda i: (i, 0)),
            pl.BlockSpec((1, N), lambda i: (0, 0)),
        ],
        out_specs=pl.BlockSpec((rt, K), lambda i: (i, 0)),
        out_shape=jax.ShapeDtypeStruct((NPAD, K), jnp.int32),
    )(s_pad, s.T, d2_pad, d2.reshape(1, -1))
    return idx


def _rotsum16(v):
    io = lax.broadcasted_iota(jnp.int32, (16,), 0)
    for sh in (8, 4, 2, 1):
        v = v + jnp.take(v, (io + sh) % 16)
    return v


CHUNK = 64


def _make_sc_agg():
    mesh = plsc.VectorSubcoreMesh(core_axis_name="c", subcore_axis_name="s")

    @functools.partial(
        pl.kernel,
        mesh=mesh,
        out_type=jax.ShapeDtypeStruct((NPAD, HW), jnp.float32),
        compiler_params=pltpu.CompilerParams(
            needs_layout_passes=False, use_tc_tiling_on_sc=False),
        scratch_types=[
            pltpu.VMEM((CPT, K), jnp.int32),
            pltpu.VMEM((CPT, 16), jnp.float32),
            pltpu.VMEM((N, 1), jnp.float32),
            pltpu.VMEM((2, K, 256), jnp.float32),
            pltpu.VMEM((2, K, 128), jnp.float32),
            pltpu.VMEM((CHUNK, HW), jnp.float32),
            pltpu.SemaphoreType.DMA,
            pltpu.SemaphoreType.DMA,
        ],
    )
    def agg_kernel(hid_hbm, idx_hbm, s128_hbm, spad_hbm, hm_hbm, out_hbm,
                   idxv, sv, hmv, rowsv, rows2v, outv, semA, semB):
        nc = 2
        wid = lax.axis_index("s") * nc + lax.axis_index("c")
        base = wid * CPT
        pltpu.sync_copy(idx_hbm.at[pl.ds(base, CPT)], idxv)
        pltpu.sync_copy(spad_hbm.at[pl.ds(base, CPT)], sv)
        pltpu.sync_copy(hm_hbm, hmv)
        io16 = lax.broadcasted_iota(jnp.int32, (16,), 0)
        sems = (semA, semB)

        def prefetch(tt, buf):
            iv = idxv[tt]
            pltpu.async_copy(hid_hbm.at[iv], rowsv.at[buf], sems[buf])
            pltpu.async_copy(s128_hbm.at[iv], rows2v.at[buf], sems[buf])

        def drain(buf):
            pltpu.make_async_copy(hid_hbm.at[idxv[0]], rowsv.at[buf], sems[buf]).wait()
            pltpu.make_async_copy(s128_hbm.at[idxv[0]], rows2v.at[buf], sems[buf]).wait()

        def process(t, tt, buf):
            iv = idxv[tt]
            drain(buf)
            s_i = sv[tt]
            hmg = plsc.load_gather(hmv, [iv, jnp.zeros((16,), jnp.int32)])
            accs = [jnp.zeros((16,), jnp.float32) for _ in range(16)]
            wvec = jnp.zeros((16,), jnp.float32)
            for k in range(K):
                dv = rows2v[buf, k, pl.ds(0, 16)] - s_i
                dtot = _rotsum16(dv * dv)
                wk = jnp.exp(dtot * jnp.float32(-1.0 / 0.09))
                wvec = jnp.where(io16 == k, wk, wvec)
                for c in range(16):
                    accs[c] = accs[c] + wk * rowsv[buf, k, pl.ds(c * 16, 16)]
            aggm = _rotsum16(wvec * hmg)
            for c in range(16):
                outv[t, pl.ds(c * 16, 16)] = accs[c]
            outv[t, pl.ds(256, 16)] = aggm

        for ch in range(CPT // CHUNK):
            if ch == 0:
                prefetch(0, 0)

            def body(u, _, ch=ch):
                t0 = 2 * u
                t1 = 2 * u + 1
                tt0 = ch * CHUNK + t0
                tt1 = tt0 + 1
                prefetch(tt1, 1)
                process(t0, tt0, 0)
                nxt = jnp.minimum(tt1 + 1, CPT - 1)
                prefetch(nxt, 0)
                process(t1, tt1, 1)
                return _

            lax.fori_loop(0, CHUNK // 2, body, 0)
            pltpu.sync_copy(outv, out_hbm.at[pl.ds(base + ch * CHUNK, CHUNK)])
        drain(0)

    return agg_kernel


def _sc_aggregate(hidden, idx, s, s_pad16, hm):
    s128 = jnp.pad(s, ((0, 0), (0, 120)))
    return _make_sc_agg()(hidden, idx, s128, s_pad16, hm)


def _final_body(agg_ref, hid_ref, hm_ref, a_ref, am_ref, b_ref, bm_ref,
                fb1_ref, w2_ref, fb2_ref, o_ref):
    agg = agg_ref[...]
    pre = jnp.dot(agg[:, 0:256], a_ref[...], preferred_element_type=jnp.float32)
    pre = pre + agg[:, 256:257] * am_ref[...]
    pre = pre + jnp.dot(hid_ref[...], b_ref[...], preferred_element_type=jnp.float32)
    pre = pre + hm_ref[...] * bm_ref[...]
    act = jnp.maximum(pre + fb1_ref[...], 0.0)
    o_ref[...] = jnp.dot(act, w2_ref[...], preferred_element_type=jnp.float32) + fb2_ref[...]


def _final_mlp(agg, hidden, hm, fW1, fb1, fW2, fb2):
    tile = 1000
    return pl.pallas_call(
        _final_body,
        grid=(N // tile,),
        in_specs=[
            pl.BlockSpec((tile, HW), lambda i: (i, 0)),
            pl.BlockSpec((tile, 256), lambda i: (i, 0)),
            pl.BlockSpec((tile, 1), lambda i: (i, 0)),
            pl.BlockSpec((256, 256), lambda i: (0, 0)),
            pl.BlockSpec((1, 256), lambda i: (0, 0)),
            pl.BlockSpec((256, 256), lambda i: (0, 0)),
            pl.BlockSpec((1, 256), lambda i: (0, 0)),
            pl.BlockSpec((1, 256), lambda i: (0, 0)),
            pl.BlockSpec((256, 256), lambda i: (0, 0)),
            pl.BlockSpec((1, 256), lambda i: (0, 0)),
        ],
        out_specs=pl.BlockSpec((tile, 256), lambda i: (i, 0)),
        out_shape=jax.ShapeDtypeStruct((N, 256), jnp.float32),
    )(agg[:N], hidden, hm,
      fW1[0:256], fW1[256:257], fW1[257:513], fW1[513:514],
      fb1.reshape(1, -1), fW2, fb2.reshape(1, -1))


def kernel(hidden_features, current_epoch, sW1, sb1, sW2, sb2, sW3, sb3, fW1, fb1, fW2, fb2):
    hm = hidden_features.mean(axis=1, keepdims=True)
    h = jnp.concatenate([hidden_features, hm], axis=-1)
    s, d2 = _spatial_mlp(h, sW1, sb1, sW2, sb2, sW3, sb3)
    idx = _knn(s, d2)

    s_pad16 = jnp.pad(s, ((0, NPAD - N), (0, 8)))
    agg = _sc_aggregate(hidden_features, idx, s, s_pad16, hm)

    out = _final_mlp(agg, hidden_features, hm, fW1, fb1, fW2, fb2)

    idx10 = idx[:N]
    start = idx10.reshape(-1).astype(jnp.int32)
    end = jnp.repeat(jnp.arange(N, dtype=jnp.int32), K)
    order = jnp.argsort(start * N + end)
    edge_index = jnp.stack([start[order], end[order]], axis=0)
    return (out, edge_index, s, 1.0)


# speedup vs baseline: 5.1624x; 1.0960x over previous
import functools

import jax
import jax.numpy as jnp
from jax import lax
from jax.experimental import pallas as pl
from jax.experimental.pallas import tpu as pltpu
from jax.experimental.pallas import tpu_sc as plsc

N = 10000
K = 16
NPAD = 10240
HW = 272
CPT = NPAD // 32


def _fold(x):
    y = x
    while y.shape[1] > 1:
        w = y.shape[1] // 2
        y = y[:, :w] + y[:, w:]
    return y


def _mlp_body(h_ref, w1_ref, b1_ref, w2_ref, b2_ref, w3_ref, b3_ref, s_ref, d2_ref):
    h = h_ref[...]
    s1 = jnp.maximum(jnp.dot(h, w1_ref[...], preferred_element_type=jnp.float32) + b1_ref[...], 0.0)
    s2 = jnp.maximum(jnp.dot(s1, w2_ref[...], preferred_element_type=jnp.float32) + b2_ref[...], 0.0)
    s3 = jnp.dot(s2, w3_ref[...], preferred_element_type=jnp.float32) + b3_ref[...]
    nrm = jnp.sqrt(_fold(s3 * s3))
    s = s3 / jnp.maximum(nrm, 1e-12)
    s_ref[...] = s
    d2_ref[...] = _fold(s * s)


def _spatial_mlp(h, sW1, sb1, sW2, sb2, sW3, sb3):
    tile = 1000
    return pl.pallas_call(
        _mlp_body,
        grid=(N // tile,),
        in_specs=[
            pl.BlockSpec((tile, 257), lambda i: (i, 0)),
            pl.BlockSpec((257, 256), lambda i: (0, 0)),
            pl.BlockSpec((1, 256), lambda i: (0, 0)),
            pl.BlockSpec((256, 256), lambda i: (0, 0)),
            pl.BlockSpec((1, 256), lambda i: (0, 0)),
            pl.BlockSpec((256, 8), lambda i: (0, 0)),
            pl.BlockSpec((1, 8), lambda i: (0, 0)),
        ],
        out_specs=[
            pl.BlockSpec((tile, 8), lambda i: (i, 0)),
            pl.BlockSpec((tile, 1), lambda i: (i, 0)),
        ],
        out_shape=[
            jax.ShapeDtypeStruct((N, 8), jnp.float32),
            jax.ShapeDtypeStruct((N, 1), jnp.float32),
        ],
    )(h, sW1, sb1.reshape(1, -1), sW2, sb2.reshape(1, -1), sW3, sb3.reshape(1, -1))


def _knn_body(sc_ref, st_ref, d2c_ref, d2r_ref, idx_ref):
    rt = sc_ref.shape[0]
    dotB = jnp.dot(sc_ref[...], st_ref[...], preferred_element_type=jnp.float32)
    Dm = d2c_ref[...] + d2r_ref[...] - 2.0 * dotB
    col = lax.broadcasted_iota(jnp.int32, (rt, N), 1)
    inf = jnp.float32(jnp.inf)
    x = Dm
    m = jnp.min(x, axis=1, keepdims=True)
    idxs = []
    for k in range(K):
        cand = jnp.where(x <= m, col, jnp.int32(2 ** 30))
        j = jnp.min(cand, axis=1)
        if k < K - 1:
            x = jnp.where(col == j[:, None], inf, x)
            m = jnp.min(x, axis=1, keepdims=True)
        idxs.append(j)
    idx_ref[...] = jnp.stack(idxs, axis=1)


def _knn(s, d2):
    rt = 512
    s_pad = jnp.pad(s, ((0, NPAD - N), (0, 0)))
    d2_pad = jnp.pad(d2, ((0, NPAD - N), (0, 0)))
    idx = pl.pallas_call(
        _knn_body,
        grid=(NPAD // rt,),
        in_specs=[
            pl.BlockSpec((rt, 8), lambda i: (i, 0)),
            pl.BlockSpec((8, N), lambda i: (0, 0)),
            pl.BlockSpec((rt, 1), lambda i: (i, 0)),
            pl.BlockSpec((1, N), lambda i: (0, 0)),
        ],
        out_specs=pl.BlockSpec((rt, K), lambda i: (i, 0)),
        out_shape=jax.ShapeDtypeStruct((NPAD, K), jnp.int32),
    )(s_pad, s.T, d2_pad, d2.reshape(1, -1))
    return idx


def _rotsum16(v):
    io = lax.broadcasted_iota(jnp.int32, (16,), 0)
    for sh in (8, 4, 2, 1):
        v = v + jnp.take(v, (io + sh) % 16)
    return v


CHUNK = 64


def _make_sc_agg():
    mesh = plsc.VectorSubcoreMesh(core_axis_name="c", subcore_axis_name="s")

    @functools.partial(
        pl.kernel,
        mesh=mesh,
        out_type=jax.ShapeDtypeStruct((NPAD, HW), jnp.float32),
        compiler_params=pltpu.CompilerParams(
            needs_layout_passes=False, use_tc_tiling_on_sc=False),
        scratch_types=[
            pltpu.VMEM((CPT, K), jnp.int32),
            pltpu.VMEM((CPT, 16), jnp.float32),
            pltpu.VMEM((N, 1), jnp.float32),
            pltpu.VMEM((2, K, 256), jnp.float32),
            pltpu.VMEM((2, K, 128), jnp.float32),
            pltpu.VMEM((CHUNK, HW), jnp.float32),
            pltpu.SemaphoreType.DMA,
            pltpu.SemaphoreType.DMA,
        ],
    )
    def agg_kernel(hid_hbm, idx_hbm, s128_hbm, spad_hbm, hm_hbm, out_hbm,
                   idxv, sv, hmv, rowsv, rows2v, outv, semA, semB):
        nc = 2
        wid = lax.axis_index("s") * nc + lax.axis_index("c")
        base = wid * CPT
        pltpu.sync_copy(idx_hbm.at[pl.ds(base, CPT)], idxv)
        pltpu.sync_copy(spad_hbm.at[pl.ds(base, CPT)], sv)
        pltpu.sync_copy(hm_hbm, hmv)
        io16 = lax.broadcasted_iota(jnp.int32, (16,), 0)
        sems = (semA, semB)

        def prefetch(tt, buf):
            iv = idxv[tt]
            pltpu.async_copy(hid_hbm.at[iv], rowsv.at[buf], sems[buf])
            pltpu.async_copy(s128_hbm.at[iv], rows2v.at[buf], sems[buf])

        def drain(buf):
            pltpu.make_async_copy(hid_hbm.at[idxv[0]], rowsv.at[buf], sems[buf]).wait()
            pltpu.make_async_copy(s128_hbm.at[idxv[0]], rows2v.at[buf], sems[buf]).wait()

        def process(t, tt, buf):
            iv = idxv[tt]
            drain(buf)
            s_i = sv[tt]
            hmg = plsc.load_gather(hmv, [iv, jnp.zeros((16,), jnp.int32)])
            accs = [jnp.zeros((16,), jnp.float32) for _ in range(16)]
            wvec = jnp.zeros((16,), jnp.float32)
            for k in range(K):
                dv = rows2v[buf, k, pl.ds(0, 16)] - s_i
                dtot = _rotsum16(dv * dv)
                wk = jnp.exp(dtot * jnp.float32(-1.0 / 0.09))
                wvec = jnp.where(io16 == k, wk, wvec)
                for c in range(16):
                    accs[c] = accs[c] + wk * rowsv[buf, k, pl.ds(c * 16, 16)]
            aggm = _rotsum16(wvec * hmg)
            for c in range(16):
                outv[t, pl.ds(c * 16, 16)] = accs[c]
            outv[t, pl.ds(256, 16)] = aggm

        for ch in range(CPT // CHUNK):
            if ch == 0:
                prefetch(0, 0)

            def body(u, _, ch=ch):
                t0 = 2 * u
                t1 = 2 * u + 1
                tt0 = ch * CHUNK + t0
                tt1 = tt0 + 1
                prefetch(tt1, 1)
                process(t0, tt0, 0)
                nxt = jnp.minimum(tt1 + 1, CPT - 1)
                prefetch(nxt, 0)
                process(t1, tt1, 1)
                return _

            lax.fori_loop(0, CHUNK // 2, body, 0)
            pltpu.sync_copy(outv, out_hbm.at[pl.ds(base + ch * CHUNK, CHUNK)])
        drain(0)

    return agg_kernel


def _sc_aggregate(hidden, idx, s, s_pad16, hm):
    s128 = jnp.pad(s, ((0, 0), (0, 120)))
    return _make_sc_agg()(hidden, idx, s128, s_pad16, hm)


def _final_body(agg_ref, hid_ref, hm_ref, a_ref, am_ref, b_ref, bm_ref,
                fb1_ref, w2_ref, fb2_ref, o_ref):
    agg = agg_ref[...]
    pre = jnp.dot(agg[:, 0:256], a_ref[...], preferred_element_type=jnp.float32)
    pre = pre + agg[:, 256:257] * am_ref[...]
    pre = pre + jnp.dot(hid_ref[...], b_ref[...], preferred_element_type=jnp.float32)
    pre = pre + hm_ref[...] * bm_ref[...]
    act = jnp.maximum(pre + fb1_ref[...], 0.0)
    o_ref[...] = jnp.dot(act, w2_ref[...], preferred_element_type=jnp.float32) + fb2_ref[...]


def _final_mlp(agg, hidden, hm, fW1, fb1, fW2, fb2):
    tile = 1000
    return pl.pallas_call(
        _final_body,
        grid=(N // tile,),
        in_specs=[
            pl.BlockSpec((tile, HW), lambda i: (i, 0)),
            pl.BlockSpec((tile, 256), lambda i: (i, 0)),
            pl.BlockSpec((tile, 1), lambda i: (i, 0)),
            pl.BlockSpec((256, 256), lambda i: (0, 0)),
            pl.BlockSpec((1, 256), lambda i: (0, 0)),
            pl.BlockSpec((256, 256), lambda i: (0, 0)),
            pl.BlockSpec((1, 256), lambda i: (0, 0)),
            pl.BlockSpec((1, 256), lambda i: (0, 0)),
            pl.BlockSpec((256, 256), lambda i: (0, 0)),
            pl.BlockSpec((1, 256), lambda i: (0, 0)),
        ],
        out_specs=pl.BlockSpec((tile, 256), lambda i: (i, 0)),
        out_shape=jax.ShapeDtypeStruct((N, 256), jnp.float32),
    )(agg[:N], hidden, hm,
      fW1[0:256], fW1[256:257], fW1[257:513], fW1[513:514],
      fb1.reshape(1, -1), fW2, fb2.reshape(1, -1))


def kernel(hidden_features, current_epoch, sW1, sb1, sW2, sb2, sW3, sb3, fW1, fb1, fW2, fb2):
    hm = hidden_features.mean(axis=1, keepdims=True)
    h = jnp.concatenate([hidden_features, hm], axis=-1)
    s, d2 = _spatial_mlp(h, sW1, sb1, sW2, sb2, sW3, sb3)
    idx = _knn(s, d2)

    s_pad16 = jnp.pad(s, ((0, NPAD - N), (0, 8)))
    agg = _sc_aggregate(hidden_features, idx, s, s_pad16, hm)

    out = _final_mlp(agg, hidden_features, hm, fW1, fb1, fW2, fb2)

    idx10 = idx[:N]
    start = idx10.reshape(-1).astype(jnp.int32)
    end = jnp.repeat(jnp.arange(N, dtype=jnp.int32), K)
    order = jnp.argsort(start * N + end)
    edge_index = jnp.stack([start[order], end[order]], axis=0)
    return (out, edge_index, s, 1.0)
